# baseline (speedup 1.0000x reference)
import functools

import jax
import jax.numpy as jnp
from jax import lax
from jax.experimental import pallas as pl
from jax.experimental.pallas import tpu as pltpu
from jax.experimental.pallas import tpu_sc as plsc

NL = 10000
NLP = 10240
NQ = 50000
NQP = 50176
D = 256

B = 512
NW = 32
QPW = NQP // NW
C = 56
NCHUNK = QPW // C


def _top3_body(ph_ref, plt_ref, idx_ref, w_ref):
    ph = ph_ref[...]
    phx, phy, phz = ph[:, 0:1], ph[:, 1:2], ph[:, 2:3]
    plx = plt_ref[0:1, :]
    ply = plt_ref[1:2, :]
    plz = plt_ref[2:3, :]
    dx = phx - plx
    dy = phy - ply
    dz = phz - plz
    d2 = dx * dx + dy * dy + dz * dz
    lane = lax.broadcasted_iota(jnp.int32, (B, NLP), 1).astype(jnp.float32)
    idxs, vals = [], []
    cur = d2
    for k in range(3):
        m = jnp.min(cur, axis=1, keepdims=True)
        eq = cur == m
        ik = jnp.min(jnp.where(eq, lane, float(NLP)), axis=1, keepdims=True)
        idxs.append(ik)
        vals.append(m)
        if k < 2:
            cur = jnp.where(eq, jnp.inf, cur)
    val = jnp.concatenate(vals, axis=1)
    w = 1.0 / jnp.maximum(val, 1e-16)
    den = jnp.sum(w, axis=1, keepdims=True)
    wn = w / den
    idx_ref[...] = jnp.concatenate(idxs, axis=1).astype(jnp.int32)
    w_ref[...] = jnp.concatenate(
        [jnp.broadcast_to(wn[:, k:k + 1], (wn.shape[0], 16)) for k in range(3)],
        axis=1)


def _top3(ph, plt):
    n = ph.shape[0]
    return pl.pallas_call(
        _top3_body,
        grid=(n // B,),
        in_specs=[
            pl.BlockSpec((B, 3), lambda i: (i, 0)),
            pl.BlockSpec((8, NLP), lambda i: (0, 0)),
        ],
        out_specs=[
            pl.BlockSpec((B, 3), lambda i: (i, 0)),
            pl.BlockSpec((B, 48), lambda i: (i, 0)),
        ],
        out_shape=[
            jax.ShapeDtypeStruct((n, 3), jnp.int32),
            jax.ShapeDtypeStruct((n, 48), jnp.float32),
        ],
    )(ph, plt)


def _sc_combine_body(qpw, x_hbm, idx_hbm, w_hbm, out_hbm,
                     idx0, idx1, w0, w1, rows0, rows1, out0, out1, sem0, sem1):
    wid = lax.axis_index("s") * 2 + lax.axis_index("c")
    qbase = wid * qpw
    nchunk = qpw // C

    def fetch(ci, idx_v, w_v, rows_v, sem):
        q0 = qbase + ci * C
        e0 = pl.multiple_of(3 * q0, 8)
        pltpu.sync_copy(idx_hbm.at[pl.ds(e0, 3 * C)], idx_v)
        pltpu.sync_copy(w_hbm.at[pl.ds(q0, C)], w_v)
        return pltpu.async_copy(x_hbm.at[idx_v], rows_v, sem)

    def compute(ci, w_v, rows_v, out_v):
        @plsc.parallel_loop(0, C, 1, unroll=2)
        def qstep(q):
            b = 3 * q
            wa = w_v[q, pl.ds(0, 16)]
            wb = w_v[q, pl.ds(16, 16)]
            wc = w_v[q, pl.ds(32, 16)]
            for f in range(D // 16):
                s = pl.ds(16 * f, 16)
                out_v[q, s] = (wa * rows_v[b, s] + wb * rows_v[b + 1, s]
                               + wc * rows_v[b + 2, s])

        q0 = qbase + ci * C
        pltpu.sync_copy(out_v, out_hbm.at[pl.ds(q0, C)])

    def pair(g, carry):
        c0 = 2 * g
        c1 = 2 * g + 1
        h0 = fetch(c0, idx0, w0, rows0, sem0)
        h1 = fetch(c1, idx1, w1, rows1, sem1)
        h0.wait()
        compute(c0, w0, rows0, out0)
        h1.wait()
        compute(c1, w1, rows1, out1)
        return carry

    lax.fori_loop(0, nchunk // 2, pair, 0)


@functools.cache
def _sc_combine(n):
    mesh = plsc.VectorSubcoreMesh(
        core_axis_name="c", subcore_axis_name="s", num_cores=2, num_subcores=16)
    return pl.kernel(
        functools.partial(_sc_combine_body, n // NW),
        out_type=jax.ShapeDtypeStruct((n, D), jnp.float32),
        mesh=mesh,
        scratch_types=[
            pltpu.VMEM((3 * C,), jnp.int32),
            pltpu.VMEM((3 * C,), jnp.int32),
            pltpu.VMEM((C, 48), jnp.float32),
            pltpu.VMEM((C, 48), jnp.float32),
            pltpu.VMEM((3 * C, D), jnp.float32),
            pltpu.VMEM((3 * C, D), jnp.float32),
            pltpu.VMEM((C, D), jnp.float32),
            pltpu.VMEM((C, D), jnp.float32),
            pltpu.SemaphoreType.DMA,
            pltpu.SemaphoreType.DMA,
        ],
    )


def kernel(x, pos_l, pos_h):
    ph = jnp.pad(pos_h, ((0, NQP - NQ), (0, 0)))
    plt = jnp.pad(pos_l.T, ((0, 5), (0, NLP - NL)),
                  constant_values=jnp.float32(jnp.inf))
    na = 35840
    idx_a, w_a = _top3(ph[:na], plt)
    out_a = _sc_combine(na)(x, idx_a.reshape(-1), w_a)
    idx_b, w_b = _top3(ph[na:], plt)
    out_b = _sc_combine(NQP - na)(x, idx_b.reshape(-1), w_b)
    return jnp.concatenate([out_a, out_b], axis=0)[:NQ]


# speedup vs baseline: 1.0100x; 1.0100x over previous
import functools

import jax
import jax.numpy as jnp
from jax import lax
from jax.experimental import pallas as pl
from jax.experimental.pallas import tpu as pltpu
from jax.experimental.pallas import tpu_sc as plsc

NL = 10000
NLP = 10240
NQ = 50000
NQP = 50176
D = 256

B = 512
NW = 32
QPW = NQP // NW
C = 56
NCHUNK = QPW // C


def _top3_body(ph_ref, plt_ref, idx_ref, w_ref):
    ph = ph_ref[...]
    phx, phy, phz = ph[:, 0:1], ph[:, 1:2], ph[:, 2:3]
    plx = plt_ref[0:1, :]
    ply = plt_ref[1:2, :]
    plz = plt_ref[2:3, :]
    dx = phx - plx
    dy = phy - ply
    dz = phz - plz
    d2 = dx * dx + dy * dy + dz * dz
    lane = lax.broadcasted_iota(jnp.int32, (B, NLP), 1).astype(jnp.float32)
    idxs, vals = [], []
    cur = d2
    for k in range(3):
        m = jnp.min(cur, axis=1, keepdims=True)
        eq = cur == m
        ik = jnp.min(jnp.where(eq, lane, float(NLP)), axis=1, keepdims=True)
        idxs.append(ik)
        vals.append(m)
        if k < 2:
            cur = jnp.where(eq, jnp.inf, cur)
    val = jnp.concatenate(vals, axis=1)
    w = 1.0 / jnp.maximum(val, 1e-16)
    den = jnp.sum(w, axis=1, keepdims=True)
    wn = w / den
    idx_ref[...] = jnp.concatenate(idxs, axis=1).astype(jnp.int32)
    w_ref[...] = jnp.concatenate(
        [jnp.broadcast_to(wn[:, k:k + 1], (wn.shape[0], 16)) for k in range(3)],
        axis=1)


def _top3(ph, plt):
    n = ph.shape[0]
    return pl.pallas_call(
        _top3_body,
        grid=(n // B,),
        in_specs=[
            pl.BlockSpec((B, 3), lambda i: (i, 0)),
            pl.BlockSpec((8, NLP), lambda i: (0, 0)),
        ],
        out_specs=[
            pl.BlockSpec((B, 3), lambda i: (i, 0)),
            pl.BlockSpec((B, 48), lambda i: (i, 0)),
        ],
        out_shape=[
            jax.ShapeDtypeStruct((n, 3), jnp.int32),
            jax.ShapeDtypeStruct((n, 48), jnp.float32),
        ],
    )(ph, plt)


def _sc_combine_body(qpw, x_hbm, idx_hbm, w_hbm, out_hbm,
                     idx0, idx1, w0, w1, rows0, rows1, out0, out1, sem0, sem1):
    wid = lax.axis_index("s") * 2 + lax.axis_index("c")
    qbase = wid * qpw
    nchunk = qpw // C

    def fetch(ci, idx_v, w_v, rows_v, sem):
        q0 = qbase + ci * C
        e0 = pl.multiple_of(3 * q0, 8)
        pltpu.sync_copy(idx_hbm.at[pl.ds(e0, 3 * C)], idx_v)
        pltpu.sync_copy(w_hbm.at[pl.ds(q0, C)], w_v)
        return pltpu.async_copy(x_hbm.at[idx_v], rows_v, sem)

    def compute(ci, w_v, rows_v, out_v):
        @plsc.parallel_loop(0, C, 1, unroll=2)
        def qstep(q):
            b = 3 * q
            wa = w_v[q, pl.ds(0, 16)]
            wb = w_v[q, pl.ds(16, 16)]
            wc = w_v[q, pl.ds(32, 16)]
            for f in range(D // 16):
                s = pl.ds(16 * f, 16)
                out_v[q, s] = (wa * rows_v[b, s] + wb * rows_v[b + 1, s]
                               + wc * rows_v[b + 2, s])

        q0 = qbase + ci * C
        pltpu.sync_copy(out_v, out_hbm.at[pl.ds(q0, C)])

    def pair(g, carry):
        c0 = 2 * g
        c1 = 2 * g + 1
        h0 = fetch(c0, idx0, w0, rows0, sem0)
        h1 = fetch(c1, idx1, w1, rows1, sem1)
        h0.wait()
        compute(c0, w0, rows0, out0)
        h1.wait()
        compute(c1, w1, rows1, out1)
        return carry

    lax.fori_loop(0, nchunk // 2, pair, 0)


@functools.cache
def _sc_combine(n):
    mesh = plsc.VectorSubcoreMesh(
        core_axis_name="c", subcore_axis_name="s", num_cores=2, num_subcores=16)
    return pl.kernel(
        functools.partial(_sc_combine_body, n // NW),
        out_type=jax.ShapeDtypeStruct((n, D), jnp.float32),
        mesh=mesh,
        scratch_types=[
            pltpu.VMEM((3 * C,), jnp.int32),
            pltpu.VMEM((3 * C,), jnp.int32),
            pltpu.VMEM((C, 48), jnp.float32),
            pltpu.VMEM((C, 48), jnp.float32),
            pltpu.VMEM((3 * C, D), jnp.float32),
            pltpu.VMEM((3 * C, D), jnp.float32),
            pltpu.VMEM((C, D), jnp.float32),
            pltpu.VMEM((C, D), jnp.float32),
            pltpu.SemaphoreType.DMA,
            pltpu.SemaphoreType.DMA,
        ],
    )


def kernel(x, pos_l, pos_h):
    ph = jnp.pad(pos_h, ((0, NQP - NQ), (0, 0)))
    plt = jnp.pad(pos_l.T, ((0, 5), (0, NLP - NL)),
                  constant_values=jnp.float32(jnp.inf))
    na = NQP // 2
    idx_a, w_a = _top3(ph[:na], plt)
    out_a = _sc_combine(na)(x, idx_a.reshape(-1), w_a)
    idx_b, w_b = _top3(ph[na:], plt)
    out_b = _sc_combine(NQP - na)(x, idx_b.reshape(-1), w_b)
    return jnp.concatenate([out_a, out_b], axis=0)[:NQ]
